# baseline (speedup 1.0000x reference)
import functools

import jax
import jax.numpy as jnp
from jax import lax
from jax.experimental import pallas as pl
from jax.experimental.pallas import tpu as pltpu
from jax.experimental.pallas import tpu_sc as plsc

EMBED_DIM = 64
CHUNK = 128

_info = plsc.get_sparse_core_info()
NUM_CORES = _info.num_cores
NUM_SUBCORES = _info.num_subcores
NUM_WORKERS = NUM_CORES * NUM_SUBCORES


def _gather_body(idx_hbm, table_hbm, out_hbm, idx_v, rows_v, sem):
    wid = lax.axis_index("s") * NUM_CORES + lax.axis_index("c")
    num_chunks = idx_hbm.shape[1]
    pltpu.sync_copy(idx_hbm.at[wid], idx_v)
    base = wid * num_chunks

    def step(j, carry):
        pltpu.async_copy(table_hbm.at[idx_v.at[j]], rows_v, sem).wait()
        pltpu.sync_copy(rows_v, out_hbm.at[base + j])
        return carry

    lax.fori_loop(0, num_chunks, step, 0)


def _gather(idx, table):
    num_chunks = idx.shape[1]
    total_chunks = NUM_WORKERS * num_chunks
    mesh = plsc.VectorSubcoreMesh(core_axis_name="c", subcore_axis_name="s")
    kfn = functools.partial(
        pl.kernel,
        mesh=mesh,
        out_type=jax.ShapeDtypeStruct((total_chunks, CHUNK, EMBED_DIM),
                                      jnp.float32),
        scratch_types=[
            pltpu.VMEM((num_chunks, CHUNK), jnp.int32),
            pltpu.VMEM((CHUNK, EMBED_DIM), jnp.float32),
            pltpu.SemaphoreType.DMA,
        ],
        compiler_params=pltpu.CompilerParams(use_tc_tiling_on_sc=False),
    )(_gather_body)
    return kfn(idx, table)


def kernel(x, table):
    b, h = x.shape
    total = b * h
    num_chunks = total // (NUM_WORKERS * CHUNK)
    idx = x.reshape(NUM_WORKERS, num_chunks, CHUNK).astype(jnp.int32)
    out = _gather(idx, table)
    return out.reshape(b, h, EMBED_DIM)


# speedup vs baseline: 1.1070x; 1.1070x over previous
import functools

import jax
import jax.numpy as jnp
from jax import lax
from jax.experimental import pallas as pl
from jax.experimental.pallas import tpu as pltpu
from jax.experimental.pallas import tpu_sc as plsc

EMBED_DIM = 64
CHUNK = 1024

_info = plsc.get_sparse_core_info()
NUM_CORES = _info.num_cores
NUM_SUBCORES = _info.num_subcores
NUM_WORKERS = NUM_CORES * NUM_SUBCORES


def _gather_body(idx_hbm, table_hbm, out_hbm, idx_v, rows_v, sem):
    wid = lax.axis_index("s") * NUM_CORES + lax.axis_index("c")
    num_chunks = idx_hbm.shape[1]
    pltpu.sync_copy(idx_hbm.at[wid], idx_v)
    base = wid * num_chunks

    def step(j, carry):
        pltpu.async_copy(table_hbm.at[idx_v.at[j]], rows_v, sem).wait()
        pltpu.sync_copy(rows_v, out_hbm.at[base + j])
        return carry

    lax.fori_loop(0, num_chunks, step, 0)


def _gather(idx, table):
    num_chunks = idx.shape[1]
    total_chunks = NUM_WORKERS * num_chunks
    mesh = plsc.VectorSubcoreMesh(core_axis_name="c", subcore_axis_name="s")
    kfn = functools.partial(
        pl.kernel,
        mesh=mesh,
        out_type=jax.ShapeDtypeStruct((total_chunks, CHUNK, EMBED_DIM),
                                      jnp.float32),
        scratch_types=[
            pltpu.VMEM((num_chunks, CHUNK), jnp.int32),
            pltpu.VMEM((CHUNK, EMBED_DIM), jnp.float32),
            pltpu.SemaphoreType.DMA,
        ],
        compiler_params=pltpu.CompilerParams(use_tc_tiling_on_sc=False),
    )(_gather_body)
    return kfn(idx, table)


def kernel(x, table):
    b, h = x.shape
    total = b * h
    num_chunks = total // (NUM_WORKERS * CHUNK)
    idx = x.reshape(NUM_WORKERS, num_chunks, CHUNK).astype(jnp.int32)
    out = _gather(idx, table)
    return out.reshape(b, h, EMBED_DIM)


# speedup vs baseline: 1.1109x; 1.0035x over previous
import functools

import jax
import jax.numpy as jnp
from jax import lax
from jax.experimental import pallas as pl
from jax.experimental.pallas import tpu as pltpu
from jax.experimental.pallas import tpu_sc as plsc

EMBED_DIM = 64
CHUNK = 640

_info = plsc.get_sparse_core_info()
NUM_CORES = _info.num_cores
NUM_SUBCORES = _info.num_subcores
NUM_WORKERS = NUM_CORES * NUM_SUBCORES


def _gather_body(idx_hbm, table_hbm, out_hbm, idx_v, rows_a, rows_b,
                 gs_a, gs_b, ws_a, ws_b):
    wid = lax.axis_index("s") * NUM_CORES + lax.axis_index("c")
    num_chunks = idx_hbm.shape[1]
    pltpu.sync_copy(idx_hbm.at[wid], idx_v)
    base = wid * num_chunks

    def gather_start(c, buf, sem):
        pltpu.async_copy(table_hbm.at[idx_v.at[c]], buf, sem)

    def gather_wait(buf, sem):
        pltpu.make_async_copy(table_hbm.at[idx_v.at[0]], buf, sem).wait()

    def write_start(c, buf, sem):
        pltpu.async_copy(buf, out_hbm.at[base + c], sem)

    def write_wait(buf, sem):
        pltpu.make_async_copy(buf, out_hbm.at[base], sem).wait()

    gather_start(0, rows_a, gs_a)
    gather_start(1, rows_b, gs_b)

    def step(i, carry):
        j = 2 * i
        gather_wait(rows_a, gs_a)
        write_start(j, rows_a, ws_a)
        gather_wait(rows_b, gs_b)
        write_start(j + 1, rows_b, ws_b)
        write_wait(rows_a, ws_a)
        gather_start(j + 2, rows_a, gs_a)
        write_wait(rows_b, ws_b)
        gather_start(j + 3, rows_b, gs_b)
        return carry

    lax.fori_loop(0, (num_chunks - 2) // 2, step, 0)

    gather_wait(rows_a, gs_a)
    write_start(num_chunks - 2, rows_a, ws_a)
    gather_wait(rows_b, gs_b)
    write_start(num_chunks - 1, rows_b, ws_b)
    write_wait(rows_a, ws_a)
    write_wait(rows_b, ws_b)


def _gather(idx, table):
    num_chunks = idx.shape[1]
    total_chunks = NUM_WORKERS * num_chunks
    mesh = plsc.VectorSubcoreMesh(core_axis_name="c", subcore_axis_name="s")
    kfn = functools.partial(
        pl.kernel,
        mesh=mesh,
        out_type=jax.ShapeDtypeStruct((total_chunks, CHUNK, EMBED_DIM),
                                      jnp.float32),
        scratch_types=[
            pltpu.VMEM((num_chunks, CHUNK), jnp.int32),
            pltpu.VMEM((CHUNK, EMBED_DIM), jnp.float32),
            pltpu.VMEM((CHUNK, EMBED_DIM), jnp.float32),
            pltpu.SemaphoreType.DMA,
            pltpu.SemaphoreType.DMA,
            pltpu.SemaphoreType.DMA,
            pltpu.SemaphoreType.DMA,
        ],
        compiler_params=pltpu.CompilerParams(use_tc_tiling_on_sc=False),
    )(_gather_body)
    return kfn(idx, table)


def kernel(x, table):
    b, h = x.shape
    total = b * h
    num_chunks = total // (NUM_WORKERS * CHUNK)
    idx = x.reshape(NUM_WORKERS, num_chunks, CHUNK).astype(jnp.int32)
    out = _gather(idx, table)
    return out.reshape(b, h, EMBED_DIM)


# speedup vs baseline: 1.3503x; 1.2155x over previous
import functools

import jax
import jax.numpy as jnp
from jax import lax
from jax.experimental import pallas as pl
from jax.experimental.pallas import tpu as pltpu
from jax.experimental.pallas import tpu_sc as plsc

BATCH = 4096
HIST = 200
EMBED_DIM = 64
PADDED_DIM = 128

_info = plsc.get_sparse_core_info()
NUM_CORES = _info.num_cores
NUM_SUBCORES = _info.num_subcores
NUM_WORKERS = NUM_CORES * NUM_SUBCORES
B_PER_W = BATCH // NUM_WORKERS


def _gather_body(x_hbm, table_hbm, out_hbm, idx_v, rows_a, rows_b,
                 gs_a, gs_b, ws_a, ws_b):
    wid = lax.axis_index("s") * NUM_CORES + lax.axis_index("c")
    b0 = wid * B_PER_W
    pltpu.sync_copy(x_hbm.at[pl.ds(b0 * HIST, B_PER_W * HIST)], idx_v)

    def gather_start(i, buf, sem):
        pltpu.async_copy(table_hbm.at[idx_v.at[pl.ds(i * HIST, HIST)]],
                         buf, sem)

    def gather_wait(buf, sem):
        pltpu.make_async_copy(table_hbm.at[idx_v.at[pl.ds(0, HIST)]],
                              buf, sem).wait()

    def write_start(i, buf, sem):
        pltpu.async_copy(buf, out_hbm.at[b0 + i], sem)

    def write_wait(buf, sem):
        pltpu.make_async_copy(buf, out_hbm.at[b0], sem).wait()

    gather_start(0, rows_a, gs_a)
    gather_start(1, rows_b, gs_b)

    def step(k, carry):
        i = 2 * k
        gather_wait(rows_a, gs_a)
        write_start(i, rows_a, ws_a)
        gather_wait(rows_b, gs_b)
        write_start(i + 1, rows_b, ws_b)
        write_wait(rows_a, ws_a)
        gather_start(i + 2, rows_a, gs_a)
        write_wait(rows_b, ws_b)
        gather_start(i + 3, rows_b, gs_b)
        return carry

    lax.fori_loop(0, (B_PER_W - 2) // 2, step, 0)

    gather_wait(rows_a, gs_a)
    write_start(B_PER_W - 2, rows_a, ws_a)
    gather_wait(rows_b, gs_b)
    write_start(B_PER_W - 1, rows_b, ws_b)
    write_wait(rows_a, ws_a)
    write_wait(rows_b, ws_b)


def _gather(x_flat, table_padded):
    mesh = plsc.VectorSubcoreMesh(core_axis_name="c", subcore_axis_name="s")
    kfn = functools.partial(
        pl.kernel,
        mesh=mesh,
        out_type=jax.ShapeDtypeStruct((BATCH, HIST, PADDED_DIM), jnp.float32),
        scratch_types=[
            pltpu.VMEM((B_PER_W * HIST,), jnp.int32),
            pltpu.VMEM((HIST, PADDED_DIM), jnp.float32),
            pltpu.VMEM((HIST, PADDED_DIM), jnp.float32),
            pltpu.SemaphoreType.DMA,
            pltpu.SemaphoreType.DMA,
            pltpu.SemaphoreType.DMA,
            pltpu.SemaphoreType.DMA,
        ],
        compiler_params=pltpu.CompilerParams(use_tc_tiling_on_sc=True),
    )(_gather_body)
    return kfn(x_flat, table_padded)


def kernel(x, table):
    x_flat = x.reshape(-1).astype(jnp.int32)
    table_padded = jnp.concatenate(
        [table, jnp.zeros((table.shape[0], PADDED_DIM - EMBED_DIM),
                          jnp.float32)], axis=1)
    return _gather(x_flat, table_padded)[..., :EMBED_DIM]
